# baseline (speedup 1.0000x reference)
import functools

import jax
import jax.numpy as jnp
from jax import lax
from jax.experimental import pallas as pl
from jax.experimental.pallas import tpu as pltpu
from jax.experimental.pallas import tpu_sc as plsc

B = 4
N1 = 65536
N2 = 16384
n1 = N1 // B
n2 = N2 // B
D = 64
K = 3
T1 = 256
T2 = 512

_NC = 2
_NS = 16
_NW = _NC * _NS
_ROWS = N1 * K
_CH = 128
_NCH = _ROWS // (_NW * _CH)


def _knn_body(p1_ref, p2_ref, col_ref, idx_ref, w_ref):
    g = pl.program_id(0)
    b = g // (n1 // T1)
    pts = p1_ref[...]
    p2t = p2_ref[0]
    colf = col_ref[...]
    p2sq = jnp.sum(p2t * p2t, axis=0, keepdims=True)
    amat = jnp.concatenate(
        [pts, jnp.ones((T1, 1), jnp.float32)], axis=1)
    bmat = jnp.concatenate([-2.0 * p2t, p2sq], axis=0)
    s = jnp.dot(amat, bmat, preferred_element_type=jnp.float32,
                precision=lax.Precision.HIGHEST)
    p1sq = jnp.sum(pts * pts, axis=1, keepdims=True)
    js = []
    ws = []
    for k in range(K):
        m = jnp.min(s, axis=1, keepdims=True)
        eq = s == m
        j = jnp.min(jnp.where(eq, colf, jnp.float32(n2)), axis=1,
                    keepdims=True)
        if k < K - 1:
            s = jnp.where(eq, jnp.float32(1e30), s)
        d2 = jnp.maximum(m + p1sq, 0.0)
        ws.append(1.0 / (jnp.sqrt(d2) + 1e-8))
        js.append(j)
    wsum = ws[0] + ws[1] + ws[2]
    w_ref[...] = (jnp.concatenate(ws, axis=1) / wsum).T
    idx = jnp.concatenate(js, axis=1).T
    idx_ref[...] = idx.astype(jnp.int32) + b * n2


@functools.lru_cache(maxsize=1)
def _make_sc_gather():
    return functools.partial(
        pl.kernel,
        mesh=plsc.VectorSubcoreMesh(core_axis_name="c", subcore_axis_name="s"),
        compiler_params=pltpu.CompilerParams(use_tc_tiling_on_sc=False),
        out_type=jax.ShapeDtypeStruct((_ROWS, D), jnp.float32),
        scratch_types=[
            pltpu.VMEM((_NCH, _CH), jnp.int32),
            pltpu.VMEM((_CH, D), jnp.float32),
            pltpu.VMEM((_CH, D), jnp.float32),
            pltpu.SemaphoreType.DMA,
            pltpu.SemaphoreType.DMA,
        ],
    )(_sc_gather_body)


def _sc_gather_body(x2_hbm, idx_hbm, out_hbm, idx_v, buf0, buf1, sem0, sem1):
    wid = lax.axis_index("s") * _NC + lax.axis_index("c")
    pltpu.sync_copy(idx_hbm.at[pl.ds(wid * _NCH, _NCH)], idx_v)
    base = wid * _NCH * _CH

    def body(jj, carry):
        j0 = jj * 2
        cp0 = pltpu.make_async_copy(x2_hbm.at[idx_v.at[j0]], buf0, sem0)
        cp0.start()
        cp1 = pltpu.make_async_copy(x2_hbm.at[idx_v.at[j0 + 1]], buf1, sem1)
        cp1.start()
        cp0.wait()
        pltpu.sync_copy(buf0, out_hbm.at[pl.ds(base + j0 * _CH, _CH)])
        cp1.wait()
        pltpu.sync_copy(buf1, out_hbm.at[pl.ds(base + (j0 + 1) * _CH, _CH)])
        return carry

    lax.fori_loop(0, _NCH // 2, body, 0)


def _mlp_body(x1_ref, f0_ref, f1_ref, f2_ref, w_ref, w1_ref, w2_ref, b_ref,
              h_ref, s_ref, ss_ref):
    g = pl.program_id(0)
    wt = w_ref[...]
    w0 = wt[0:1, :].T
    w1 = wt[1:2, :].T
    w2 = wt[2:3, :].T
    interp = (f0_ref[...] * w0 + f1_ref[...] * w1
              + f2_ref[...] * w2)
    h = (jnp.dot(x1_ref[...], w1_ref[...], preferred_element_type=jnp.float32,
                 precision=lax.Precision.HIGHEST)
         + jnp.dot(interp, w2_ref[...], preferred_element_type=jnp.float32,
                   precision=lax.Precision.HIGHEST)
         + b_ref[...])
    h_ref[...] = h
    hr = h.reshape(8, T2 // 8, D)
    ps = jnp.sum(hr, axis=1)
    pss = jnp.sum(hr * hr, axis=1)

    @pl.when(g == 0)
    def _():
        s_ref[...] = ps
        ss_ref[...] = pss

    @pl.when(g > 0)
    def _():
        s_ref[...] = s_ref[...] + ps
        ss_ref[...] = ss_ref[...] + pss


def _bn_body(h_ref, s_ref, ss_ref, gm_ref, bt_ref, o_ref):
    s = jnp.sum(s_ref[...], axis=0, keepdims=True)
    ss = jnp.sum(ss_ref[...], axis=0, keepdims=True)
    mu = s / N1
    var = ss / N1 - mu * mu
    inv = lax.rsqrt(var + 1e-5)
    h = h_ref[...]
    o_ref[...] = jnp.maximum(gm_ref[...] * ((h - mu) * inv) + bt_ref[...], 0.0)


def kernel(p1, x1, o1, p2, x2, o2, W, b, gamma, beta):
    p2t = p2.reshape(B, n2, 3).transpose(0, 2, 1)
    nt1 = n1 // T1
    idx_t, w_t = pl.pallas_call(
        _knn_body,
        grid=(B * nt1,),
        in_specs=[
            pl.BlockSpec((T1, 3), lambda g: (g, 0)),
            pl.BlockSpec((1, 3, n2), lambda g: (g // nt1, 0, 0)),
            pl.BlockSpec((1, n2), lambda g: (0, 0)),
        ],
        out_specs=[
            pl.BlockSpec((K, T1), lambda g: (0, g)),
            pl.BlockSpec((K, T1), lambda g: (0, g)),
        ],
        out_shape=[
            jax.ShapeDtypeStruct((K, N1), jnp.int32),
            jax.ShapeDtypeStruct((K, N1), jnp.float32),
        ],
    )(p1, p2t, jnp.arange(n2, dtype=jnp.float32)[None, :])

    idx2d = idx_t.reshape(_ROWS // _CH, _CH)
    feats = _make_sc_gather()(x2, idx2d)

    nt2 = N1 // T2
    row_spec = pl.BlockSpec((T2, D), lambda g: (g, 0))
    f_specs = [
        pl.BlockSpec((T2, D), lambda g, k=k: (k * nt2 + g, 0)) for k in range(K)
    ]
    full64 = pl.BlockSpec((D, D), lambda g: (0, 0))
    stat_spec = pl.BlockSpec((8, D), lambda g: (0, 0))
    h, s, ss = pl.pallas_call(
        _mlp_body,
        grid=(nt2,),
        in_specs=[
            row_spec, *f_specs,
            pl.BlockSpec((K, T2), lambda g: (0, g)),
            full64, full64,
            pl.BlockSpec((1, D), lambda g: (0, 0)),
        ],
        out_specs=[row_spec, stat_spec, stat_spec],
        out_shape=[
            jax.ShapeDtypeStruct((N1, D), jnp.float32),
            jax.ShapeDtypeStruct((8, D), jnp.float32),
            jax.ShapeDtypeStruct((8, D), jnp.float32),
        ],
    )(x1, feats, feats, feats, w_t, W[:D], W[D:], b.reshape(1, D))

    x = pl.pallas_call(
        _bn_body,
        grid=(nt2,),
        in_specs=[
            row_spec, stat_spec, stat_spec,
            pl.BlockSpec((1, D), lambda g: (0, 0)),
            pl.BlockSpec((1, D), lambda g: (0, 0)),
        ],
        out_specs=row_spec,
        out_shape=jax.ShapeDtypeStruct((N1, D), jnp.float32),
    )(h, s, ss, gamma.reshape(1, D), beta.reshape(1, D))

    return (p1, x, o1)


# speedup vs baseline: 1.4098x; 1.4098x over previous
import functools

import jax
import jax.numpy as jnp
from jax import lax
from jax.experimental import pallas as pl
from jax.experimental.pallas import tpu as pltpu
from jax.experimental.pallas import tpu_sc as plsc

B = 4
N1 = 65536
N2 = 16384
n1 = N1 // B
n2 = N2 // B
D = 64
K = 3
T1 = 256
T2 = 512

_NC = 2
_NS = 16
_NW = _NC * _NS
_ROWS = N1 * K
_CH = 128
_NCH = _ROWS // (_NW * _CH)


def _knn_body(p1_ref, p2_ref, col_ref, idx_ref, w_ref):
    g = pl.program_id(0)
    b = g // (n1 // T1)
    pts = p1_ref[...]
    p2t = p2_ref[0]
    colf = col_ref[...]
    p2sq = (p2t[0:1] * p2t[0:1] + p2t[1:2] * p2t[1:2]
            + p2t[2:3] * p2t[2:3])
    m2 = -2.0 * pts
    s = (p2sq + m2[:, 0:1] * p2t[0:1]
         + m2[:, 1:2] * p2t[1:2]
         + m2[:, 2:3] * p2t[2:3])
    p1sq = jnp.sum(pts * pts, axis=1, keepdims=True)
    js = []
    ws = []
    for k in range(K):
        m = jnp.min(s, axis=1, keepdims=True)
        eq = s == m
        j = jnp.min(jnp.where(eq, colf, jnp.float32(n2)), axis=1,
                    keepdims=True)
        if k < K - 1:
            s = jnp.where(eq, jnp.float32(1e30), s)
        d2 = jnp.maximum(m + p1sq, 0.0)
        ws.append(1.0 / (jnp.sqrt(d2) + 1e-8))
        js.append(j)
    wsum = ws[0] + ws[1] + ws[2]
    w_ref[...] = (jnp.concatenate(ws, axis=1) / wsum).T
    idx = jnp.concatenate(js, axis=1).T
    idx_ref[...] = idx.astype(jnp.int32) + b * n2


@functools.lru_cache(maxsize=1)
def _make_sc_gather():
    return functools.partial(
        pl.kernel,
        mesh=plsc.VectorSubcoreMesh(core_axis_name="c", subcore_axis_name="s"),
        compiler_params=pltpu.CompilerParams(use_tc_tiling_on_sc=False),
        out_type=jax.ShapeDtypeStruct((_ROWS, D), jnp.float32),
        scratch_types=[
            pltpu.VMEM((_NCH, _CH), jnp.int32),
            pltpu.VMEM((_CH, D), jnp.float32),
            pltpu.VMEM((_CH, D), jnp.float32),
            pltpu.SemaphoreType.DMA,
            pltpu.SemaphoreType.DMA,
        ],
    )(_sc_gather_body)


def _sc_gather_body(x2_hbm, idx_hbm, out_hbm, idx_v, buf0, buf1, sem0, sem1):
    wid = lax.axis_index("s") * _NC + lax.axis_index("c")
    pltpu.sync_copy(idx_hbm.at[pl.ds(wid * _NCH, _NCH)], idx_v)
    base = wid * _NCH * _CH

    def body(jj, carry):
        j0 = jj * 2
        cp0 = pltpu.make_async_copy(x2_hbm.at[idx_v.at[j0]], buf0, sem0)
        cp0.start()
        cp1 = pltpu.make_async_copy(x2_hbm.at[idx_v.at[j0 + 1]], buf1, sem1)
        cp1.start()
        cp0.wait()
        pltpu.sync_copy(buf0, out_hbm.at[pl.ds(base + j0 * _CH, _CH)])
        cp1.wait()
        pltpu.sync_copy(buf1, out_hbm.at[pl.ds(base + (j0 + 1) * _CH, _CH)])
        return carry

    lax.fori_loop(0, _NCH // 2, body, 0)


def _mlp_body(x1_ref, f0_ref, f1_ref, f2_ref, w_ref, w1_ref, w2_ref, b_ref,
              h_ref, s_ref, ss_ref):
    g = pl.program_id(0)
    wt = w_ref[...]
    w0 = wt[0:1, :].T
    w1 = wt[1:2, :].T
    w2 = wt[2:3, :].T
    interp = (f0_ref[...] * w0 + f1_ref[...] * w1
              + f2_ref[...] * w2)
    h = (jnp.dot(x1_ref[...], w1_ref[...], preferred_element_type=jnp.float32,
                 precision=lax.Precision.HIGHEST)
         + jnp.dot(interp, w2_ref[...], preferred_element_type=jnp.float32,
                   precision=lax.Precision.HIGHEST)
         + b_ref[...])
    h_ref[...] = h
    hr = h.reshape(8, T2 // 8, D)
    ps = jnp.sum(hr, axis=1)
    pss = jnp.sum(hr * hr, axis=1)

    @pl.when(g == 0)
    def _():
        s_ref[...] = ps
        ss_ref[...] = pss

    @pl.when(g > 0)
    def _():
        s_ref[...] = s_ref[...] + ps
        ss_ref[...] = ss_ref[...] + pss


def _bn_body(h_ref, s_ref, ss_ref, gm_ref, bt_ref, o_ref):
    s = jnp.sum(s_ref[...], axis=0, keepdims=True)
    ss = jnp.sum(ss_ref[...], axis=0, keepdims=True)
    mu = s / N1
    var = ss / N1 - mu * mu
    inv = lax.rsqrt(var + 1e-5)
    h = h_ref[...]
    o_ref[...] = jnp.maximum(gm_ref[...] * ((h - mu) * inv) + bt_ref[...], 0.0)


def kernel(p1, x1, o1, p2, x2, o2, W, b, gamma, beta):
    p2t = p2.reshape(B, n2, 3).transpose(0, 2, 1)
    nt1 = n1 // T1
    idx_t, w_t = pl.pallas_call(
        _knn_body,
        grid=(B * nt1,),
        in_specs=[
            pl.BlockSpec((T1, 3), lambda g: (g, 0)),
            pl.BlockSpec((1, 3, n2), lambda g: (g // nt1, 0, 0)),
            pl.BlockSpec((1, n2), lambda g: (0, 0)),
        ],
        out_specs=[
            pl.BlockSpec((K, T1), lambda g: (0, g)),
            pl.BlockSpec((K, T1), lambda g: (0, g)),
        ],
        out_shape=[
            jax.ShapeDtypeStruct((K, N1), jnp.int32),
            jax.ShapeDtypeStruct((K, N1), jnp.float32),
        ],
    )(p1, p2t, jnp.arange(n2, dtype=jnp.float32)[None, :])

    idx2d = idx_t.reshape(_ROWS // _CH, _CH)
    feats = _make_sc_gather()(x2, idx2d)

    nt2 = N1 // T2
    row_spec = pl.BlockSpec((T2, D), lambda g: (g, 0))
    f_specs = [
        pl.BlockSpec((T2, D), lambda g, k=k: (k * nt2 + g, 0)) for k in range(K)
    ]
    full64 = pl.BlockSpec((D, D), lambda g: (0, 0))
    stat_spec = pl.BlockSpec((8, D), lambda g: (0, 0))
    h, s, ss = pl.pallas_call(
        _mlp_body,
        grid=(nt2,),
        in_specs=[
            row_spec, *f_specs,
            pl.BlockSpec((K, T2), lambda g: (0, g)),
            full64, full64,
            pl.BlockSpec((1, D), lambda g: (0, 0)),
        ],
        out_specs=[row_spec, stat_spec, stat_spec],
        out_shape=[
            jax.ShapeDtypeStruct((N1, D), jnp.float32),
            jax.ShapeDtypeStruct((8, D), jnp.float32),
            jax.ShapeDtypeStruct((8, D), jnp.float32),
        ],
    )(x1, feats, feats, feats, w_t, W[:D], W[D:], b.reshape(1, D))

    x = pl.pallas_call(
        _bn_body,
        grid=(nt2,),
        in_specs=[
            row_spec, stat_spec, stat_spec,
            pl.BlockSpec((1, D), lambda g: (0, 0)),
            pl.BlockSpec((1, D), lambda g: (0, 0)),
        ],
        out_specs=row_spec,
        out_shape=jax.ShapeDtypeStruct((N1, D), jnp.float32),
    )(h, s, ss, gamma.reshape(1, D), beta.reshape(1, D))

    return (p1, x, o1)


# speedup vs baseline: 1.4157x; 1.0042x over previous
import functools

import jax
import jax.numpy as jnp
from jax import lax
from jax.experimental import pallas as pl
from jax.experimental.pallas import tpu as pltpu
from jax.experimental.pallas import tpu_sc as plsc

B = 4
N1 = 65536
N2 = 16384
n1 = N1 // B
n2 = N2 // B
D = 64
K = 3
T1 = 512
T2 = 512

_NC = 2
_NS = 16
_NW = _NC * _NS
_ROWS = N1 * K
_CH = 128
_NCH = _ROWS // (_NW * _CH)


def _knn_body(p1_ref, p2_ref, col_ref, idx_ref, w_ref):
    g = pl.program_id(0)
    b = g // (n1 // T1)
    pts = p1_ref[...]
    p2t = p2_ref[0]
    colf = col_ref[...]
    p2sq = (p2t[0:1] * p2t[0:1] + p2t[1:2] * p2t[1:2]
            + p2t[2:3] * p2t[2:3])
    m2 = -2.0 * pts
    s = (p2sq + m2[:, 0:1] * p2t[0:1]
         + m2[:, 1:2] * p2t[1:2]
         + m2[:, 2:3] * p2t[2:3])
    p1sq = jnp.sum(pts * pts, axis=1, keepdims=True)
    js = []
    ws = []
    for k in range(K):
        m = jnp.min(s, axis=1, keepdims=True)
        eq = s == m
        j = jnp.min(jnp.where(eq, colf, jnp.float32(n2)), axis=1,
                    keepdims=True)
        if k < K - 1:
            s = jnp.where(eq, jnp.float32(1e30), s)
        d2 = jnp.maximum(m + p1sq, 0.0)
        ws.append(1.0 / (jnp.sqrt(d2) + 1e-8))
        js.append(j)
    wsum = ws[0] + ws[1] + ws[2]
    w_ref[...] = (jnp.concatenate(ws, axis=1) / wsum).T
    idx = jnp.concatenate(js, axis=1).T
    idx_ref[...] = idx.astype(jnp.int32) + b * n2


@functools.lru_cache(maxsize=1)
def _make_sc_gather():
    return functools.partial(
        pl.kernel,
        mesh=plsc.VectorSubcoreMesh(core_axis_name="c", subcore_axis_name="s"),
        compiler_params=pltpu.CompilerParams(use_tc_tiling_on_sc=False),
        out_type=jax.ShapeDtypeStruct((_ROWS, D), jnp.float32),
        scratch_types=[
            pltpu.VMEM((_NCH, _CH), jnp.int32),
            pltpu.VMEM((_CH, D), jnp.float32),
            pltpu.VMEM((_CH, D), jnp.float32),
            pltpu.SemaphoreType.DMA,
            pltpu.SemaphoreType.DMA,
        ],
    )(_sc_gather_body)


def _sc_gather_body(x2_hbm, idx_hbm, out_hbm, idx_v, buf0, buf1, sem0, sem1):
    wid = lax.axis_index("s") * _NC + lax.axis_index("c")
    pltpu.sync_copy(idx_hbm.at[pl.ds(wid * _NCH, _NCH)], idx_v)
    base = wid * _NCH * _CH

    def body(jj, carry):
        j0 = jj * 2
        cp0 = pltpu.make_async_copy(x2_hbm.at[idx_v.at[j0]], buf0, sem0)
        cp0.start()
        cp1 = pltpu.make_async_copy(x2_hbm.at[idx_v.at[j0 + 1]], buf1, sem1)
        cp1.start()
        cp0.wait()
        pltpu.sync_copy(buf0, out_hbm.at[pl.ds(base + j0 * _CH, _CH)])
        cp1.wait()
        pltpu.sync_copy(buf1, out_hbm.at[pl.ds(base + (j0 + 1) * _CH, _CH)])
        return carry

    lax.fori_loop(0, _NCH // 2, body, 0)


def _mlp_body(x1_ref, f0_ref, f1_ref, f2_ref, w_ref, w1_ref, w2_ref, b_ref,
              h_ref, s_ref, ss_ref):
    g = pl.program_id(0)
    wt = w_ref[...]
    w0 = wt[0:1, :].T
    w1 = wt[1:2, :].T
    w2 = wt[2:3, :].T
    interp = (f0_ref[...] * w0 + f1_ref[...] * w1
              + f2_ref[...] * w2)
    h = (jnp.dot(x1_ref[...], w1_ref[...], preferred_element_type=jnp.float32,
                 precision=lax.Precision.HIGHEST)
         + jnp.dot(interp, w2_ref[...], preferred_element_type=jnp.float32,
                   precision=lax.Precision.HIGHEST)
         + b_ref[...])
    h_ref[...] = h
    hr = h.reshape(8, T2 // 8, D)
    ps = jnp.sum(hr, axis=1)
    pss = jnp.sum(hr * hr, axis=1)

    @pl.when(g == 0)
    def _():
        s_ref[...] = ps
        ss_ref[...] = pss

    @pl.when(g > 0)
    def _():
        s_ref[...] = s_ref[...] + ps
        ss_ref[...] = ss_ref[...] + pss


def _bn_body(h_ref, s_ref, ss_ref, gm_ref, bt_ref, o_ref):
    s = jnp.sum(s_ref[...], axis=0, keepdims=True)
    ss = jnp.sum(ss_ref[...], axis=0, keepdims=True)
    mu = s / N1
    var = ss / N1 - mu * mu
    inv = lax.rsqrt(var + 1e-5)
    h = h_ref[...]
    o_ref[...] = jnp.maximum(gm_ref[...] * ((h - mu) * inv) + bt_ref[...], 0.0)


def kernel(p1, x1, o1, p2, x2, o2, W, b, gamma, beta):
    p2t = p2.reshape(B, n2, 3).transpose(0, 2, 1)
    nt1 = n1 // T1
    idx_t, w_t = pl.pallas_call(
        _knn_body,
        grid=(B * nt1,),
        in_specs=[
            pl.BlockSpec((T1, 3), lambda g: (g, 0)),
            pl.BlockSpec((1, 3, n2), lambda g: (g // nt1, 0, 0)),
            pl.BlockSpec((1, n2), lambda g: (0, 0)),
        ],
        out_specs=[
            pl.BlockSpec((K, T1), lambda g: (0, g)),
            pl.BlockSpec((K, T1), lambda g: (0, g)),
        ],
        out_shape=[
            jax.ShapeDtypeStruct((K, N1), jnp.int32),
            jax.ShapeDtypeStruct((K, N1), jnp.float32),
        ],
    )(p1, p2t, jnp.arange(n2, dtype=jnp.float32)[None, :])

    idx2d = idx_t.reshape(_ROWS // _CH, _CH)
    feats = _make_sc_gather()(x2, idx2d)

    nt2 = N1 // T2
    row_spec = pl.BlockSpec((T2, D), lambda g: (g, 0))
    f_specs = [
        pl.BlockSpec((T2, D), lambda g, k=k: (k * nt2 + g, 0)) for k in range(K)
    ]
    full64 = pl.BlockSpec((D, D), lambda g: (0, 0))
    stat_spec = pl.BlockSpec((8, D), lambda g: (0, 0))
    h, s, ss = pl.pallas_call(
        _mlp_body,
        grid=(nt2,),
        in_specs=[
            row_spec, *f_specs,
            pl.BlockSpec((K, T2), lambda g: (0, g)),
            full64, full64,
            pl.BlockSpec((1, D), lambda g: (0, 0)),
        ],
        out_specs=[row_spec, stat_spec, stat_spec],
        out_shape=[
            jax.ShapeDtypeStruct((N1, D), jnp.float32),
            jax.ShapeDtypeStruct((8, D), jnp.float32),
            jax.ShapeDtypeStruct((8, D), jnp.float32),
        ],
    )(x1, feats, feats, feats, w_t, W[:D], W[D:], b.reshape(1, D))

    x = pl.pallas_call(
        _bn_body,
        grid=(nt2,),
        in_specs=[
            row_spec, stat_spec, stat_spec,
            pl.BlockSpec((1, D), lambda g: (0, 0)),
            pl.BlockSpec((1, D), lambda g: (0, 0)),
        ],
        out_specs=row_spec,
        out_shape=jax.ShapeDtypeStruct((N1, D), jnp.float32),
    )(h, s, ss, gamma.reshape(1, D), beta.reshape(1, D))

    return (p1, x, o1)
